# Initial kernel scaffold; baseline (speedup 1.0000x reference)
import functools
import jax
import jax.numpy as jnp
from jax import lax
from jax.experimental import pallas as pl
from jax.experimental.pallas import tpu as pltpu
from jax.experimental.pallas import tpu_sc as plsc

N = 10000
E = 320000
IN_CH = 128
HID = 64
HEADS = 8
OUT_CH = 40

NC, NS, L = 2, 16, 16
NW = NC * NS
CB = 128
NCHUNKS = E // CB
CPW = -(-NCHUNKS // NW)
RPS = N // NS

SROW1, DW1 = 80, 8
SROW2, DW2 = 48, 8

_NEG = -3.0e38


def _lrelu(x):
    return jnp.maximum(x, 0.2 * x)


def _tc1_body(x_ref, w1_ref, asrc_ref, adst_ref, s_ref, d_ref, m_ref):
    i = pl.program_id(0)
    h = jnp.dot(x_ref[...], w1_ref[...], preferred_element_type=jnp.float32)
    a_s = jnp.dot(h, asrc_ref[...], preferred_element_type=jnp.float32)
    a_d = jnp.dot(h, adst_ref[...], preferred_element_type=jnp.float32)
    blk = h.shape[0]
    s_ref[...] = jnp.concatenate(
        [h, a_s, jnp.zeros((blk, SROW1 - HID - HEADS), jnp.float32)], axis=1)
    d_ref[...] = a_d
    m16 = jnp.concatenate([jnp.max(a_s, axis=0), jnp.max(a_d, axis=0)],
                          axis=0).reshape(1, 16)

    @pl.when(i == 0)
    def _():
        m_ref[...] = m16

    @pl.when(i > 0)
    def _():
        m_ref[...] = jnp.maximum(m_ref[...], m16)


def _tc1(x, W1, Asrc, Adst):
    BN = 1000
    grid = N // BN
    return pl.pallas_call(
        _tc1_body,
        grid=(grid,),
        in_specs=[
            pl.BlockSpec((BN, IN_CH), lambda i: (i, 0)),
            pl.BlockSpec((IN_CH, HID), lambda i: (0, 0)),
            pl.BlockSpec((HID, HEADS), lambda i: (0, 0)),
            pl.BlockSpec((HID, HEADS), lambda i: (0, 0)),
        ],
        out_specs=[
            pl.BlockSpec((BN, SROW1), lambda i: (i, 0)),
            pl.BlockSpec((BN, DW1), lambda i: (i, 0)),
            pl.BlockSpec((1, 16), lambda i: (0, 0)),
        ],
        out_shape=[
            jax.ShapeDtypeStruct((N, SROW1), jnp.float32),
            jax.ShapeDtypeStruct((N, DW1), jnp.float32),
            jax.ShapeDtypeStruct((1, 16), jnp.float32),
        ],
    )(x, W1, Asrc, Adst)


def _edge_kernel(srow, dw, phase_fn):
    mesh = plsc.VectorSubcoreMesh(core_axis_name="c", subcore_axis_name="s",
                                  num_cores=NC, num_subcores=NS)

    @functools.partial(
        pl.kernel,
        out_type=jax.ShapeDtypeStruct((NC, N, srow), jnp.float32),
        mesh=mesh,
        scratch_types=[
            pltpu.VMEM((CB, srow), jnp.float32),
            pltpu.VMEM((CB, dw), jnp.float32),
            pltpu.VMEM((CB * HEADS,), jnp.float32),
            pltpu.VMEM((CB,), jnp.int32),
            pltpu.VMEM((CB,), jnp.int32),
            pltpu.VMEM((16,), jnp.float32),
            pltpu.VMEM_SHARED((N, srow), jnp.float32),
            pltpu.SemaphoreType.DMA,
            pltpu.SemaphoreType.DMA,
        ],
    )
    def k(s_hbm, d_hbm, m_hbm, src_hbm, dst_hbm, z_hbm, out_hbm,
          sbuf, dbuf, ebuf, sidx, didx, mvec, accum, sem1, sem2):
        cid = lax.axis_index("c")
        sid = lax.axis_index("s")
        w = sid * NC + cid
        r0 = sid * RPS
        pltpu.sync_copy(z_hbm.at[pl.ds(r0, RPS)], accum.at[pl.ds(r0, RPS)])
        pltpu.sync_copy(m_hbm, mvec)
        plsc.subcore_barrier()

        def body(kk, carry):
            cidx = w + NW * kk

            @pl.when(cidx < NCHUNKS)
            def _():
                base = cidx * CB
                pltpu.sync_copy(src_hbm.at[pl.ds(base, CB)], sidx)
                pltpu.sync_copy(dst_hbm.at[pl.ds(base, CB)], didx)
                pltpu.async_copy(s_hbm.at[sidx], sbuf, sem1).wait()
                pltpu.async_copy(d_hbm.at[didx], dbuf, sem2).wait()
                phase_fn(sbuf, dbuf, ebuf, mvec)
                pltpu.sync_copy(sbuf, accum.at[didx], add=True)

            return carry

        lax.fori_loop(0, CPW, body, 0)
        plsc.subcore_barrier()
        pltpu.sync_copy(accum.at[pl.ds(r0, RPS)],
                        out_hbm.at[cid, pl.ds(r0, RPS)])

    return k


def _phase_l1(sbuf, dbuf, ebuf, mvec):
    i16 = lax.iota(jnp.int32, 16)
    lane8 = i16 & 7
    hi = (i16 >= 8).astype(jnp.int32)
    ms = plsc.load_gather(mvec, [lane8])
    md = plsc.load_gather(mvec, [lane8 + 8])
    cpair = _lrelu(ms + md)

    def p1(p, carry):
        rows = 2 * p + hi
        a_s = plsc.load_gather(sbuf, [rows, HID + lane8])
        a_d = plsc.load_gather(dbuf, [rows, lane8])
        al = a_s + a_d
        ebuf[pl.ds(16 * p, 16)] = jnp.exp(_lrelu(al) - cpair)
        return carry

    lax.fori_loop(0, CB // 2, p1, 0)

    def p2(b, carry):
        eb = HEADS * b
        for j in range(4):
            ev = plsc.load_gather(ebuf, [eb + 2 * j + hi])
            sbuf[b, pl.ds(16 * j, 16)] = sbuf[b, pl.ds(16 * j, 16)] * ev
        sbuf[b, pl.ds(HID, 16)] = plsc.load_gather(ebuf, [eb + lane8])
        return carry

    lax.fori_loop(0, CB, p2, 0)


def _phase_l2(sbuf, dbuf, ebuf, mvec):
    i16 = lax.iota(jnp.int32, 16)
    z16 = i16 * 0
    cs = _lrelu(plsc.load_gather(mvec, [z16]) +
                plsc.load_gather(mvec, [z16 + 1]))

    def p1(p, carry):
        rows = 16 * p + i16
        a_s = plsc.load_gather(sbuf, [rows, z16 + OUT_CH])
        a_d = plsc.load_gather(dbuf, [rows, z16])
        ebuf[pl.ds(16 * p, 16)] = jnp.exp(_lrelu(a_s + a_d) - cs)
        return carry

    lax.fori_loop(0, CB // 16, p1, 0)

    def p2(b, carry):
        ev = plsc.load_gather(ebuf, [z16 + b])
        for j in range(3):
            val = sbuf[b, pl.ds(16 * j, 16)] * ev
            if j == 2:
                val = jnp.where(i16 == 8, ev, val)
            sbuf[b, pl.ds(16 * j, 16)] = val
        return carry

    lax.fori_loop(0, CB, p2, 0)


def _tc2_body(p_ref, b1_ref, w2_ref, e8_ref, v2_ref, s_ref, d_ref, m_ref):
    i = pl.program_id(0)
    p0 = p_ref[0]
    p1 = p_ref[1]
    num = p0[:, :HID] + p1[:, :HID]
    den8 = p0[:, HID:HID + HEADS] + p1[:, HID:HID + HEADS]
    den64 = jnp.dot(den8, e8_ref[...], preferred_element_type=jnp.float32)
    h1 = num / (den64 + 1e-16) + b1_ref[...]
    h1 = jnp.where(h1 > 0, h1, jnp.exp(jnp.minimum(h1, 0.0)) - 1.0)
    h2 = jnp.dot(h1, w2_ref[...], preferred_element_type=jnp.float32)
    v2 = v2_ref[...]
    a_s = jnp.sum(h2 * v2[0:1, :], axis=1, keepdims=True)
    a_d = jnp.sum(h2 * v2[1:2, :], axis=1, keepdims=True)
    blk = h2.shape[0]
    s_ref[...] = jnp.concatenate(
        [h2, a_s, jnp.zeros((blk, SROW2 - OUT_CH - 1), jnp.float32)], axis=1)
    d_ref[...] = jnp.concatenate(
        [a_d, jnp.zeros((blk, DW2 - 1), jnp.float32)], axis=1)
    m16 = jnp.full((1, 16), _NEG, jnp.float32)
    lanes = lax.broadcasted_iota(jnp.int32, (1, 16), 1)
    m16 = jnp.where(lanes == 0, jnp.max(a_s), m16)
    m16 = jnp.where(lanes == 1, jnp.max(a_d), m16)

    @pl.when(i == 0)
    def _():
        m_ref[...] = m16

    @pl.when(i > 0)
    def _():
        m_ref[...] = jnp.maximum(m_ref[...], m16)


def _tc2(P1, b1, W2, E8, V2):
    BN = 1000
    grid = N // BN
    return pl.pallas_call(
        _tc2_body,
        grid=(grid,),
        in_specs=[
            pl.BlockSpec((NC, BN, SROW1), lambda i: (0, i, 0)),
            pl.BlockSpec((1, HID), lambda i: (0, 0)),
            pl.BlockSpec((HID, OUT_CH), lambda i: (0, 0)),
            pl.BlockSpec((HEADS, HID), lambda i: (0, 0)),
            pl.BlockSpec((2, OUT_CH), lambda i: (0, 0)),
        ],
        out_specs=[
            pl.BlockSpec((BN, SROW2), lambda i: (i, 0)),
            pl.BlockSpec((BN, DW2), lambda i: (i, 0)),
            pl.BlockSpec((1, 16), lambda i: (0, 0)),
        ],
        out_shape=[
            jax.ShapeDtypeStruct((N, SROW2), jnp.float32),
            jax.ShapeDtypeStruct((N, DW2), jnp.float32),
            jax.ShapeDtypeStruct((1, 16), jnp.float32),
        ],
    )(P1, b1, W2, E8, V2)


def _tc3_body(p_ref, b2_ref, o_ref):
    p0 = p_ref[0]
    p1 = p_ref[1]
    num = p0[:, :OUT_CH] + p1[:, :OUT_CH]
    den = p0[:, OUT_CH:OUT_CH + 1] + p1[:, OUT_CH:OUT_CH + 1]
    o = num / (den + 1e-16) + b2_ref[...]
    m = jnp.max(o, axis=1, keepdims=True)
    o_ref[...] = o - m - jnp.log(jnp.sum(jnp.exp(o - m), axis=1,
                                         keepdims=True))


def _tc3(P2, b2):
    BN = 1000
    grid = N // BN
    return pl.pallas_call(
        _tc3_body,
        grid=(grid,),
        in_specs=[
            pl.BlockSpec((NC, BN, SROW2), lambda i: (0, i, 0)),
            pl.BlockSpec((1, OUT_CH), lambda i: (0, 0)),
        ],
        out_specs=pl.BlockSpec((BN, OUT_CH), lambda i: (i, 0)),
        out_shape=jax.ShapeDtypeStruct((N, OUT_CH), jnp.float32),
    )(P2, b2)


def kernel(x, edge_index, W1, att_src1, att_dst1, b1, W2, att_src2, att_dst2,
           b2):
    src = edge_index[0]
    dst = edge_index[1]

    rows = jnp.arange(HID)
    Asrc = jnp.zeros((HID, HEADS), jnp.float32).at[rows, rows // 8].set(
        att_src1.reshape(-1))
    Adst = jnp.zeros((HID, HEADS), jnp.float32).at[rows, rows // 8].set(
        att_dst1.reshape(-1))
    E8 = jnp.zeros((HEADS, HID), jnp.float32).at[rows // 8, rows].set(1.0)
    V2 = jnp.concatenate([att_src2, att_dst2], axis=0)

    S1, D1, M1 = _tc1(x, W1, Asrc, Adst)
    sc1 = _edge_kernel(SROW1, DW1, _phase_l1)
    P1 = sc1(S1, D1, M1.reshape(16), src, dst,
             jnp.zeros((N, SROW1), jnp.float32))

    S2, D2, M2 = _tc2(P1, b1.reshape(1, HID), W2, E8, V2)
    sc2 = _edge_kernel(SROW2, DW2, _phase_l2)
    P2 = sc2(S2, D2, M2.reshape(16), src, dst,
             jnp.zeros((N, SROW2), jnp.float32))

    return _tc3(P2, b2.reshape(1, OUT_CH))


# speedup vs baseline: 48.9921x; 48.9921x over previous
import functools
import jax
import jax.numpy as jnp
from jax import lax
from jax.experimental import pallas as pl
from jax.experimental.pallas import tpu as pltpu
from jax.experimental.pallas import tpu_sc as plsc

N = 10000
E = 320000
IN_CH = 128
HID = 64
HEADS = 8
OUT_CH = 40

NC, NS, L = 2, 16, 16
NW = NC * NS
CB = 128
NCHUNKS = E // CB
CPW = -(-NCHUNKS // NW)
RPS = 624
RTAIL = N - NS * RPS

SROW1, DW1 = 80, 8
SROW2, DW2 = 48, 8

_NEG = -3.0e38


def _lrelu(x):
    return jnp.maximum(x, 0.2 * x)


def _tc1_body(x_ref, w1_ref, asrc_ref, adst_ref, s_ref, d_ref, m_ref):
    i = pl.program_id(0)
    h = jnp.dot(x_ref[...], w1_ref[...], preferred_element_type=jnp.float32)
    a_s = jnp.dot(h, asrc_ref[...], preferred_element_type=jnp.float32)
    a_d = jnp.dot(h, adst_ref[...], preferred_element_type=jnp.float32)
    blk = h.shape[0]
    s_ref[...] = jnp.concatenate(
        [h, a_s, jnp.zeros((blk, SROW1 - HID - HEADS), jnp.float32)], axis=1)
    d_ref[...] = a_d
    m16 = jnp.concatenate([jnp.max(a_s, axis=0), jnp.max(a_d, axis=0)],
                          axis=0).reshape(1, 16)

    @pl.when(i == 0)
    def _():
        m_ref[...] = m16

    @pl.when(i > 0)
    def _():
        m_ref[...] = jnp.maximum(m_ref[...], m16)

    @pl.when(i == pl.num_programs(0) - 1)
    def _():
        m = m_ref[...]
        c = _lrelu(m[:, :HEADS] + m[:, HEADS:])
        m_ref[...] = jnp.concatenate([c, c], axis=1)


def _tc1(x, W1, Asrc, Adst):
    BN = 1000
    grid = N // BN
    return pl.pallas_call(
        _tc1_body,
        grid=(grid,),
        in_specs=[
            pl.BlockSpec((BN, IN_CH), lambda i: (i, 0)),
            pl.BlockSpec((IN_CH, HID), lambda i: (0, 0)),
            pl.BlockSpec((HID, HEADS), lambda i: (0, 0)),
            pl.BlockSpec((HID, HEADS), lambda i: (0, 0)),
        ],
        out_specs=[
            pl.BlockSpec((BN, SROW1), lambda i: (i, 0)),
            pl.BlockSpec((BN, DW1), lambda i: (i, 0)),
            pl.BlockSpec((1, 16), lambda i: (0, 0)),
        ],
        out_shape=[
            jax.ShapeDtypeStruct((N, SROW1), jnp.float32),
            jax.ShapeDtypeStruct((N, DW1), jnp.float32),
            jax.ShapeDtypeStruct((1, 16), jnp.float32),
        ],
    )(x, W1, Asrc, Adst)


def _edge_kernel(srow, dw, phase_fn):
    mesh = plsc.VectorSubcoreMesh(core_axis_name="c", subcore_axis_name="s",
                                  num_cores=NC, num_subcores=NS)

    @functools.partial(
        pl.kernel,
        out_type=jax.ShapeDtypeStruct((NC, N, srow), jnp.float32),
        mesh=mesh,
        scratch_types=[
            pltpu.VMEM((CB, srow), jnp.float32),
            pltpu.VMEM((CB, dw), jnp.float32),
            pltpu.VMEM((CB * HEADS,), jnp.float32),
            pltpu.VMEM((CB,), jnp.int32),
            pltpu.VMEM((CB,), jnp.int32),
            pltpu.VMEM((16,), jnp.float32),
            pltpu.VMEM_SHARED((N, srow), jnp.float32),
            pltpu.SemaphoreType.DMA,
            pltpu.SemaphoreType.DMA,
        ],
        compiler_params=pltpu.CompilerParams(use_tc_tiling_on_sc=False,
                                             needs_layout_passes=False),
    )
    def k(s_hbm, d_hbm, m_hbm, src_hbm, dst_hbm, z_hbm, out_hbm,
          sbuf, dbuf, ebuf, sidx, didx, cbuf, accum, sem1, sem2):
        cid = lax.axis_index("c")
        sid = lax.axis_index("s")
        w = sid * NC + cid
        r0 = sid * RPS
        pltpu.sync_copy(z_hbm.at[pl.ds(r0, RPS)], accum.at[pl.ds(r0, RPS)])

        @pl.when(sid == 0)
        def _():
            pltpu.sync_copy(z_hbm.at[pl.ds(NS * RPS, RTAIL)],
                            accum.at[pl.ds(NS * RPS, RTAIL)])

        pltpu.sync_copy(m_hbm, cbuf)
        plsc.subcore_barrier()

        def body(kk, carry):
            cidx = w + NW * kk

            @pl.when(cidx < NCHUNKS)
            def _():
                base = cidx * CB
                pltpu.sync_copy(src_hbm.at[pl.ds(base, CB)], sidx)
                pltpu.sync_copy(dst_hbm.at[pl.ds(base, CB)], didx)
                pltpu.async_copy(s_hbm.at[sidx], sbuf, sem1).wait()
                pltpu.async_copy(d_hbm.at[didx], dbuf, sem2).wait()
                phase_fn(sbuf, dbuf, ebuf, cbuf)
                pltpu.sync_copy(sbuf, accum.at[didx], add=True)

            return carry

        lax.fori_loop(0, CPW, body, 0)
        plsc.subcore_barrier()
        pltpu.sync_copy(accum.at[pl.ds(r0, RPS)],
                        out_hbm.at[cid, pl.ds(r0, RPS)])

        @pl.when(sid == 0)
        def _():
            pltpu.sync_copy(accum.at[pl.ds(NS * RPS, RTAIL)],
                            out_hbm.at[cid, pl.ds(NS * RPS, RTAIL)])

    return k


def _phase_l1(sbuf, dbuf, ebuf, cbuf):
    def p1(p, carry):
        i16 = lax.iota(jnp.int32, 16)
        lane8 = i16 & 7
        hi = (i16 >= 8).astype(jnp.int32)
        cpair = cbuf[pl.ds(0, 16)]
        rows = 2 * p + hi
        a_s = plsc.load_gather(sbuf, [rows, HID + lane8])
        a_d = plsc.load_gather(dbuf, [rows, lane8])
        al = a_s + a_d
        ebuf[pl.ds(16 * p, 16)] = jnp.exp(_lrelu(al) - cpair)
        return carry

    lax.fori_loop(0, CB // 2, p1, 0)

    def p2(b, carry):
        i16 = lax.iota(jnp.int32, 16)
        lane8 = i16 & 7
        hi = (i16 >= 8).astype(jnp.int32)
        eb = HEADS * b
        for j in range(4):
            ev = plsc.load_gather(ebuf, [eb + 2 * j + hi])
            sbuf[b, pl.ds(16 * j, 16)] = sbuf[b, pl.ds(16 * j, 16)] * ev
        sbuf[b, pl.ds(HID, 16)] = plsc.load_gather(ebuf, [eb + lane8])
        return carry

    lax.fori_loop(0, CB, p2, 0)


def _phase_l2(sbuf, dbuf, ebuf, cbuf):
    def p1(p, carry):
        i16 = lax.iota(jnp.int32, 16)
        z16 = i16 * 0
        cs = cbuf[pl.ds(0, 16)]
        rows = 16 * p + i16
        a_s = plsc.load_gather(sbuf, [rows, z16 + OUT_CH])
        a_d = plsc.load_gather(dbuf, [rows, z16])
        ebuf[pl.ds(16 * p, 16)] = jnp.exp(_lrelu(a_s + a_d) - cs)
        return carry

    lax.fori_loop(0, CB // 16, p1, 0)

    def p2(b, carry):
        i16 = lax.iota(jnp.int32, 16)
        z16 = i16 * 0
        ev = plsc.load_gather(ebuf, [z16 + b])
        for j in range(3):
            val = sbuf[b, pl.ds(16 * j, 16)] * ev
            if j == 2:
                val = jnp.where(i16 == 8, ev, val)
            sbuf[b, pl.ds(16 * j, 16)] = val
        return carry

    lax.fori_loop(0, CB, p2, 0)


def _tc2_body(p_ref, b1_ref, w2_ref, e8_ref, v2_ref, s_ref, d_ref, m_ref):
    i = pl.program_id(0)
    p0 = p_ref[0]
    p1 = p_ref[1]
    num = p0[:, :HID] + p1[:, :HID]
    den8 = p0[:, HID:HID + HEADS] + p1[:, HID:HID + HEADS]
    den64 = jnp.dot(den8, e8_ref[...], preferred_element_type=jnp.float32)
    h1 = num / (den64 + 1e-16) + b1_ref[...]
    h1 = jnp.where(h1 > 0, h1, jnp.exp(jnp.minimum(h1, 0.0)) - 1.0)
    h2 = jnp.dot(h1, w2_ref[...], preferred_element_type=jnp.float32)
    v2 = v2_ref[...]
    a_s = jnp.sum(h2 * v2[0:1, :], axis=1, keepdims=True)
    a_d = jnp.sum(h2 * v2[1:2, :], axis=1, keepdims=True)
    blk = h2.shape[0]
    s_ref[...] = jnp.concatenate(
        [h2, a_s, jnp.zeros((blk, SROW2 - OUT_CH - 1), jnp.float32)], axis=1)
    d_ref[...] = jnp.concatenate(
        [a_d, jnp.zeros((blk, DW2 - 1), jnp.float32)], axis=1)
    m16 = jnp.full((1, 16), _NEG, jnp.float32)
    lanes = lax.broadcasted_iota(jnp.int32, (1, 16), 1)
    m16 = jnp.where(lanes == 0, jnp.max(a_s), m16)
    m16 = jnp.where(lanes == 1, jnp.max(a_d), m16)

    @pl.when(i == 0)
    def _():
        m_ref[...] = m16

    @pl.when(i > 0)
    def _():
        m_ref[...] = jnp.maximum(m_ref[...], m16)

    @pl.when(i == pl.num_programs(0) - 1)
    def _():
        m = m_ref[...]
        ms = jnp.sum(jnp.where(lanes == 0, m, 0.0), axis=1, keepdims=True)
        md = jnp.sum(jnp.where(lanes == 1, m, 0.0), axis=1, keepdims=True)
        m_ref[...] = jnp.broadcast_to(_lrelu(ms + md), (1, 16))


def _tc2(P1, b1, W2, E8, V2):
    BN = 1000
    grid = N // BN
    return pl.pallas_call(
        _tc2_body,
        grid=(grid,),
        in_specs=[
            pl.BlockSpec((NC, BN, SROW1), lambda i: (0, i, 0)),
            pl.BlockSpec((1, HID), lambda i: (0, 0)),
            pl.BlockSpec((HID, OUT_CH), lambda i: (0, 0)),
            pl.BlockSpec((HEADS, HID), lambda i: (0, 0)),
            pl.BlockSpec((2, OUT_CH), lambda i: (0, 0)),
        ],
        out_specs=[
            pl.BlockSpec((BN, SROW2), lambda i: (i, 0)),
            pl.BlockSpec((BN, DW2), lambda i: (i, 0)),
            pl.BlockSpec((1, 16), lambda i: (0, 0)),
        ],
        out_shape=[
            jax.ShapeDtypeStruct((N, SROW2), jnp.float32),
            jax.ShapeDtypeStruct((N, DW2), jnp.float32),
            jax.ShapeDtypeStruct((1, 16), jnp.float32),
        ],
    )(P1, b1, W2, E8, V2)


def _tc3_body(p_ref, b2_ref, o_ref):
    p0 = p_ref[0]
    p1 = p_ref[1]
    num = p0[:, :OUT_CH] + p1[:, :OUT_CH]
    den = p0[:, OUT_CH:OUT_CH + 1] + p1[:, OUT_CH:OUT_CH + 1]
    o = num / (den + 1e-16) + b2_ref[...]
    m = jnp.max(o, axis=1, keepdims=True)
    o_ref[...] = o - m - jnp.log(jnp.sum(jnp.exp(o - m), axis=1,
                                         keepdims=True))


def _tc3(P2, b2):
    BN = 1000
    grid = N // BN
    return pl.pallas_call(
        _tc3_body,
        grid=(grid,),
        in_specs=[
            pl.BlockSpec((NC, BN, SROW2), lambda i: (0, i, 0)),
            pl.BlockSpec((1, OUT_CH), lambda i: (0, 0)),
        ],
        out_specs=pl.BlockSpec((BN, OUT_CH), lambda i: (i, 0)),
        out_shape=jax.ShapeDtypeStruct((N, OUT_CH), jnp.float32),
    )(P2, b2)


def kernel(x, edge_index, W1, att_src1, att_dst1, b1, W2, att_src2, att_dst2,
           b2):
    src = edge_index[0]
    dst = edge_index[1]

    rows = jnp.arange(HID)
    Asrc = jnp.zeros((HID, HEADS), jnp.float32).at[rows, rows // 8].set(
        att_src1.reshape(-1))
    Adst = jnp.zeros((HID, HEADS), jnp.float32).at[rows, rows // 8].set(
        att_dst1.reshape(-1))
    E8 = jnp.zeros((HEADS, HID), jnp.float32).at[rows // 8, rows].set(1.0)
    V2 = jnp.concatenate([att_src2, att_dst2], axis=0)

    S1, D1, M1 = _tc1(x, W1, Asrc, Adst)
    sc1 = _edge_kernel(SROW1, DW1, _phase_l1)
    P1 = sc1(S1, D1, M1.reshape(16), src, dst,
             jnp.zeros((N, SROW1), jnp.float32))

    S2, D2, M2 = _tc2(P1, b1.reshape(1, HID), W2, E8, V2)
    sc2 = _edge_kernel(SROW2, DW2, _phase_l2)
    P2 = sc2(S2, D2, M2.reshape(16), src, dst,
             jnp.zeros((N, SROW2), jnp.float32))

    return _tc3(P2, b2.reshape(1, OUT_CH))


# speedup vs baseline: 74.4234x; 1.5191x over previous
import functools
import jax
import jax.numpy as jnp
from jax import lax
from jax.experimental import pallas as pl
from jax.experimental.pallas import tpu as pltpu
from jax.experimental.pallas import tpu_sc as plsc

N = 10000
E = 320000
IN_CH = 128
HID = 64
HEADS = 8
OUT_CH = 40

NC, NS, L = 2, 16, 16
NW = NC * NS
CB = 128
NCHUNKS = E // CB
CPW = -(-NCHUNKS // NW)
RPS = 624
RTAIL = N - NS * RPS

SROW1, DW1 = 80, 8
SROW2, DW2 = 48, 8

_NEG = -3.0e38


def _lrelu(x):
    return jnp.maximum(x, 0.2 * x)


def _tc1_body(x_ref, w1_ref, asrc_ref, adst_ref, s_ref, d_ref, m_ref):
    i = pl.program_id(0)
    h = jnp.dot(x_ref[...], w1_ref[...], preferred_element_type=jnp.float32)
    a_s = jnp.dot(h, asrc_ref[...], preferred_element_type=jnp.float32)
    a_d = jnp.dot(h, adst_ref[...], preferred_element_type=jnp.float32)
    blk = h.shape[0]
    s_ref[...] = jnp.concatenate(
        [h, a_s, jnp.zeros((blk, SROW1 - HID - HEADS), jnp.float32)], axis=1)
    d_ref[...] = a_d
    m16 = jnp.concatenate([jnp.max(a_s, axis=0), jnp.max(a_d, axis=0)],
                          axis=0).reshape(1, 16)

    @pl.when(i == 0)
    def _():
        m_ref[...] = m16

    @pl.when(i > 0)
    def _():
        m_ref[...] = jnp.maximum(m_ref[...], m16)

    @pl.when(i == pl.num_programs(0) - 1)
    def _():
        m = m_ref[...]
        c = _lrelu(m[:, :HEADS] + m[:, HEADS:])
        m_ref[...] = jnp.concatenate([c, c], axis=1)


def _tc1(x, W1, Asrc, Adst):
    BN = 1000
    grid = N // BN
    return pl.pallas_call(
        _tc1_body,
        grid=(grid,),
        in_specs=[
            pl.BlockSpec((BN, IN_CH), lambda i: (i, 0)),
            pl.BlockSpec((IN_CH, HID), lambda i: (0, 0)),
            pl.BlockSpec((HID, HEADS), lambda i: (0, 0)),
            pl.BlockSpec((HID, HEADS), lambda i: (0, 0)),
        ],
        out_specs=[
            pl.BlockSpec((BN, SROW1), lambda i: (i, 0)),
            pl.BlockSpec((BN, DW1), lambda i: (i, 0)),
            pl.BlockSpec((1, 16), lambda i: (0, 0)),
        ],
        out_shape=[
            jax.ShapeDtypeStruct((N, SROW1), jnp.float32),
            jax.ShapeDtypeStruct((N, DW1), jnp.float32),
            jax.ShapeDtypeStruct((1, 16), jnp.float32),
        ],
    )(x, W1, Asrc, Adst)


SUP = 2
SB = SUP * CB
NSUP = NCHUNKS // SUP
T2 = 20


def _edge_kernel(srow, dw, phase_fn):
    mesh = plsc.VectorSubcoreMesh(core_axis_name="c", subcore_axis_name="s",
                                  num_cores=NC, num_subcores=NS)

    @functools.partial(
        pl.kernel,
        out_type=jax.ShapeDtypeStruct((NC, N, srow), jnp.float32),
        mesh=mesh,
        scratch_types=[
            pltpu.VMEM((SB, srow), jnp.float32),
            pltpu.VMEM((SB, srow), jnp.float32),
            pltpu.VMEM((SB, dw), jnp.float32),
            pltpu.VMEM((SB, dw), jnp.float32),
            pltpu.VMEM((SB * HEADS,), jnp.float32),
            pltpu.VMEM((16,), jnp.float32),
            pltpu.VMEM((SUP, CB), jnp.int32),
            pltpu.VMEM((SUP, CB), jnp.int32),
            pltpu.VMEM((SUP, CB), jnp.int32),
            pltpu.VMEM((SUP, CB), jnp.int32),
            pltpu.VMEM_SHARED((N, srow), jnp.float32),
            pltpu.SemaphoreType.DMA,
            pltpu.SemaphoreType.DMA,
            pltpu.SemaphoreType.DMA,
            pltpu.SemaphoreType.DMA,
            pltpu.SemaphoreType.DMA,
            pltpu.SemaphoreType.DMA,
        ],
        compiler_params=pltpu.CompilerParams(use_tc_tiling_on_sc=False,
                                             needs_layout_passes=False),
    )
    def k(s_hbm, d_hbm, m_hbm, src2_hbm, dst2_hbm, z_hbm, out_hbm,
          sbuf0, sbuf1, dbuf0, dbuf1, ebuf, cbuf,
          sidx0, sidx1, didx0, didx1, accum,
          semS0, semS1, semD0, semD1, semW0, semW1):
        sbufs = (sbuf0, sbuf1)
        dbufs = (dbuf0, dbuf1)
        sidxs = (sidx0, sidx1)
        didxs = (didx0, didx1)
        semSs = (semS0, semS1)
        semDs = (semD0, semD1)
        semWs = (semW0, semW1)

        cid = lax.axis_index("c")
        sid = lax.axis_index("s")
        w = sid * NC + cid
        r0 = sid * RPS
        pltpu.sync_copy(z_hbm.at[pl.ds(r0, RPS)], accum.at[pl.ds(r0, RPS)])

        @pl.when(sid == 0)
        def _():
            pltpu.sync_copy(z_hbm.at[pl.ds(NS * RPS, RTAIL)],
                            accum.at[pl.ds(NS * RPS, RTAIL)])

        pltpu.sync_copy(m_hbm, cbuf)
        plsc.subcore_barrier()

        def fire(sup, P):
            pltpu.sync_copy(src2_hbm.at[pl.ds(sup * SUP, SUP)], sidxs[P])
            pltpu.sync_copy(dst2_hbm.at[pl.ds(sup * SUP, SUP)], didxs[P])
            for c in range(SUP):
                pltpu.async_copy(s_hbm.at[sidxs[P].at[c]],
                                 sbufs[P].at[pl.ds(c * CB, CB)], semSs[P])
                pltpu.async_copy(d_hbm.at[didxs[P].at[c]],
                                 dbufs[P].at[pl.ds(c * CB, CB)], semDs[P])

        def drain(dummy_hbm, buf, sem):
            pltpu.make_async_copy(dummy_hbm, buf, sem).wait()

        def process(P):
            drain(s_hbm.at[pl.ds(0, SB)], sbufs[P], semSs[P])
            drain(d_hbm.at[pl.ds(0, SB)], dbufs[P], semDs[P])
            phase_fn(sbufs[P], dbufs[P], ebuf, cbuf)
            for c in range(SUP):
                pltpu.async_copy(sbufs[P].at[pl.ds(c * CB, CB)],
                                 accum.at[didxs[P].at[c]], semWs[P], add=True)

        fire(w, 0)

        def body(t2, carry):
            for toff in (0, 1):
                P = toff
                Q = 1 - toff
                t = 2 * t2 + toff
                s_nxt = w + NW * (t + 1)
                s_cur = w + NW * t

                def prefetch():
                    if toff == 1:
                        drain(s_hbm.at[pl.ds(0, SB)], sbufs[Q], semWs[Q])
                    else:
                        @pl.when(t2 >= 1)
                        def _():
                            drain(s_hbm.at[pl.ds(0, SB)], sbufs[Q], semWs[Q])
                    fire(s_nxt, Q)

                pl.when(s_nxt < NSUP)(prefetch)
                pl.when(s_cur < NSUP)(lambda: process(P))
            return carry

        lax.fori_loop(0, T2, body, 0)
        drain(s_hbm.at[pl.ds(0, SB)], sbufs[0], semWs[0])
        drain(s_hbm.at[pl.ds(0, SB)], sbufs[1], semWs[1])
        plsc.subcore_barrier()
        pltpu.sync_copy(accum.at[pl.ds(r0, RPS)],
                        out_hbm.at[cid, pl.ds(r0, RPS)])

        @pl.when(sid == 0)
        def _():
            pltpu.sync_copy(accum.at[pl.ds(NS * RPS, RTAIL)],
                            out_hbm.at[cid, pl.ds(NS * RPS, RTAIL)])

    return k


def _phase_l1(sbuf, dbuf, ebuf, cbuf):
    def p1(p, carry):
        i16 = lax.iota(jnp.int32, 16)
        lane8 = i16 & 7
        hi = (i16 >= 8).astype(jnp.int32)
        cpair = cbuf[pl.ds(0, 16)]
        rows = 2 * p + hi
        a_s = plsc.load_gather(sbuf, [rows, HID + lane8])
        a_d = plsc.load_gather(dbuf, [rows, lane8])
        al = a_s + a_d
        ebuf[pl.ds(16 * p, 16)] = jnp.exp(_lrelu(al) - cpair)
        return carry

    lax.fori_loop(0, SB // 2, p1, 0, unroll=4)

    def p2(b, carry):
        i16 = lax.iota(jnp.int32, 16)
        lane8 = i16 & 7
        hi = (i16 >= 8).astype(jnp.int32)
        eb = HEADS * b
        for j in range(4):
            ev = plsc.load_gather(ebuf, [eb + 2 * j + hi])
            sbuf[b, pl.ds(16 * j, 16)] = sbuf[b, pl.ds(16 * j, 16)] * ev
        sbuf[b, pl.ds(HID, 16)] = plsc.load_gather(ebuf, [eb + lane8])
        return carry

    lax.fori_loop(0, SB, p2, 0, unroll=4)


def _phase_l2(sbuf, dbuf, ebuf, cbuf):
    def p1(p, carry):
        i16 = lax.iota(jnp.int32, 16)
        z16 = i16 * 0
        cs = cbuf[pl.ds(0, 16)]
        rows = 16 * p + i16
        a_s = plsc.load_gather(sbuf, [rows, z16 + OUT_CH])
        a_d = plsc.load_gather(dbuf, [rows, z16])
        ebuf[pl.ds(16 * p, 16)] = jnp.exp(_lrelu(a_s + a_d) - cs)
        return carry

    lax.fori_loop(0, SB // 16, p1, 0, unroll=4)

    def p2(b, carry):
        i16 = lax.iota(jnp.int32, 16)
        z16 = i16 * 0
        ev = plsc.load_gather(ebuf, [z16 + b])
        for j in range(3):
            val = sbuf[b, pl.ds(16 * j, 16)] * ev
            if j == 2:
                val = jnp.where(i16 == 8, ev, val)
            sbuf[b, pl.ds(16 * j, 16)] = val
        return carry

    lax.fori_loop(0, SB, p2, 0, unroll=4)


def _tc2_body(p_ref, b1_ref, w2_ref, e8_ref, v2_ref, s_ref, d_ref, m_ref):
    i = pl.program_id(0)
    p0 = p_ref[0]
    p1 = p_ref[1]
    num = p0[:, :HID] + p1[:, :HID]
    den8 = p0[:, HID:HID + HEADS] + p1[:, HID:HID + HEADS]
    den64 = jnp.dot(den8, e8_ref[...], preferred_element_type=jnp.float32)
    h1 = num / (den64 + 1e-16) + b1_ref[...]
    h1 = jnp.where(h1 > 0, h1, jnp.exp(jnp.minimum(h1, 0.0)) - 1.0)
    h2 = jnp.dot(h1, w2_ref[...], preferred_element_type=jnp.float32)
    v2 = v2_ref[...]
    a_s = jnp.sum(h2 * v2[0:1, :], axis=1, keepdims=True)
    a_d = jnp.sum(h2 * v2[1:2, :], axis=1, keepdims=True)
    blk = h2.shape[0]
    s_ref[...] = jnp.concatenate(
        [h2, a_s, jnp.zeros((blk, SROW2 - OUT_CH - 1), jnp.float32)], axis=1)
    d_ref[...] = jnp.concatenate(
        [a_d, jnp.zeros((blk, DW2 - 1), jnp.float32)], axis=1)
    m16 = jnp.full((1, 16), _NEG, jnp.float32)
    lanes = lax.broadcasted_iota(jnp.int32, (1, 16), 1)
    m16 = jnp.where(lanes == 0, jnp.max(a_s), m16)
    m16 = jnp.where(lanes == 1, jnp.max(a_d), m16)

    @pl.when(i == 0)
    def _():
        m_ref[...] = m16

    @pl.when(i > 0)
    def _():
        m_ref[...] = jnp.maximum(m_ref[...], m16)

    @pl.when(i == pl.num_programs(0) - 1)
    def _():
        m = m_ref[...]
        ms = jnp.sum(jnp.where(lanes == 0, m, 0.0), axis=1, keepdims=True)
        md = jnp.sum(jnp.where(lanes == 1, m, 0.0), axis=1, keepdims=True)
        m_ref[...] = jnp.broadcast_to(_lrelu(ms + md), (1, 16))


def _tc2(P1, b1, W2, E8, V2):
    BN = 1000
    grid = N // BN
    return pl.pallas_call(
        _tc2_body,
        grid=(grid,),
        in_specs=[
            pl.BlockSpec((NC, BN, SROW1), lambda i: (0, i, 0)),
            pl.BlockSpec((1, HID), lambda i: (0, 0)),
            pl.BlockSpec((HID, OUT_CH), lambda i: (0, 0)),
            pl.BlockSpec((HEADS, HID), lambda i: (0, 0)),
            pl.BlockSpec((2, OUT_CH), lambda i: (0, 0)),
        ],
        out_specs=[
            pl.BlockSpec((BN, SROW2), lambda i: (i, 0)),
            pl.BlockSpec((BN, DW2), lambda i: (i, 0)),
            pl.BlockSpec((1, 16), lambda i: (0, 0)),
        ],
        out_shape=[
            jax.ShapeDtypeStruct((N, SROW2), jnp.float32),
            jax.ShapeDtypeStruct((N, DW2), jnp.float32),
            jax.ShapeDtypeStruct((1, 16), jnp.float32),
        ],
    )(P1, b1, W2, E8, V2)


def _tc3_body(p_ref, b2_ref, o_ref):
    p0 = p_ref[0]
    p1 = p_ref[1]
    num = p0[:, :OUT_CH] + p1[:, :OUT_CH]
    den = p0[:, OUT_CH:OUT_CH + 1] + p1[:, OUT_CH:OUT_CH + 1]
    o = num / (den + 1e-16) + b2_ref[...]
    m = jnp.max(o, axis=1, keepdims=True)
    o_ref[...] = o - m - jnp.log(jnp.sum(jnp.exp(o - m), axis=1,
                                         keepdims=True))


def _tc3(P2, b2):
    BN = 1000
    grid = N // BN
    return pl.pallas_call(
        _tc3_body,
        grid=(grid,),
        in_specs=[
            pl.BlockSpec((NC, BN, SROW2), lambda i: (0, i, 0)),
            pl.BlockSpec((1, OUT_CH), lambda i: (0, 0)),
        ],
        out_specs=pl.BlockSpec((BN, OUT_CH), lambda i: (i, 0)),
        out_shape=jax.ShapeDtypeStruct((N, OUT_CH), jnp.float32),
    )(P2, b2)


def kernel(x, edge_index, W1, att_src1, att_dst1, b1, W2, att_src2, att_dst2,
           b2):
    src = edge_index[0]
    dst = edge_index[1]

    rows = jnp.arange(HID)
    Asrc = jnp.zeros((HID, HEADS), jnp.float32).at[rows, rows // 8].set(
        att_src1.reshape(-1))
    Adst = jnp.zeros((HID, HEADS), jnp.float32).at[rows, rows // 8].set(
        att_dst1.reshape(-1))
    E8 = jnp.zeros((HEADS, HID), jnp.float32).at[rows // 8, rows].set(1.0)
    V2 = jnp.concatenate([att_src2, att_dst2], axis=0)

    src2 = src.reshape(NCHUNKS, CB)
    dst2 = dst.reshape(NCHUNKS, CB)

    S1, D1, M1 = _tc1(x, W1, Asrc, Adst)
    sc1 = _edge_kernel(SROW1, DW1, _phase_l1)
    P1 = sc1(S1, D1, M1.reshape(16), src2, dst2,
             jnp.zeros((N, SROW1), jnp.float32))

    S2, D2, M2 = _tc2(P1, b1.reshape(1, HID), W2, E8, V2)
    sc2 = _edge_kernel(SROW2, DW2, _phase_l2)
    P2 = sc2(S2, D2, M2.reshape(16), src2, dst2,
             jnp.zeros((N, SROW2), jnp.float32))

    return _tc3(P2, b2.reshape(1, OUT_CH))
